# Initial kernel scaffold; baseline (speedup 1.0000x reference)
import jax
import jax.numpy as jnp
from jax.experimental import pallas as pl


def kernel(coord, label, params):
    raise NotImplementedError("write your pallas kernel here")


# speedup vs baseline: 1.0304x; 1.0304x over previous
import functools
import jax
import jax.numpy as jnp
from jax import lax
from jax.experimental import pallas as pl
from jax.experimental.pallas import tpu as pltpu


def _fps_body(xyz_ref, out_ref, *, npoint):
    B, _, N = xyz_ref.shape
    x0 = xyz_ref[:, 0, :]
    x1 = xyz_ref[:, 1, :]
    x2 = xyz_ref[:, 2, :]
    iota_n = (lax.broadcasted_iota(jnp.int32, (B, N), 1)
              + 0 * lax.broadcasted_iota(jnp.int32, (B, N), 0)
              ).astype(jnp.float32)
    iota_s = (lax.broadcasted_iota(jnp.int32, (B, npoint), 1)
              + 0 * lax.broadcasted_iota(jnp.int32, (B, npoint), 0)
              ).astype(jnp.float32)

    def body(i, st):
        cent, dist, far = st
        cent = jnp.where(iota_s == i.astype(jnp.float32), far, cent)
        mask = iota_n == far
        cx = jnp.sum(jnp.where(mask, x0, 0.0), axis=1, keepdims=True)
        cy = jnp.sum(jnp.where(mask, x1, 0.0), axis=1, keepdims=True)
        cz = jnp.sum(jnp.where(mask, x2, 0.0), axis=1, keepdims=True)
        d0 = x0 - cx
        d1 = x1 - cy
        d2 = x2 - cz
        d = d0 * d0 + d1 * d1 + d2 * d2
        dist = jnp.minimum(dist, d)
        m = jnp.max(dist, axis=1, keepdims=True)
        far_new = jnp.min(jnp.where(dist == m, iota_n, float(N)), axis=1,
                          keepdims=True)
        return cent, dist, far_new

    init = (x0[:, :npoint] * 0.0,
            x0 * 0.0 + 1e10,
            x0[:, :1] * 0.0)
    cent, _, _ = lax.fori_loop(0, npoint, body, init)
    out_ref[...] = cent.astype(jnp.int32)


def fps_pallas(xyz_t, npoint):
    B, _, N = xyz_t.shape
    return pl.pallas_call(
        functools.partial(_fps_body, npoint=npoint),
        out_shape=jax.ShapeDtypeStruct((B, npoint), jnp.int32),
    )(xyz_t)


def _square_distance(src, dst):
    return (jnp.sum(src ** 2, -1)[:, :, None] + jnp.sum(dst ** 2, -1)[:, None, :]
            - 2.0 * jnp.einsum('bnc,bmc->bnm', src, dst))


def _index_points(points, idx):
    return jax.vmap(lambda p, i: p[i])(points, idx)


def _query_ball_point(radius, nsample, xyz, new_xyz):
    B, N, _ = xyz.shape
    S = new_xyz.shape[1]
    sqrdists = _square_distance(new_xyz, xyz)
    group_idx = jnp.broadcast_to(jnp.arange(N, dtype=jnp.int32), (B, S, N))
    group_idx = jnp.where(sqrdists > radius ** 2, N, group_idx)
    group_idx = jnp.sort(group_idx, axis=-1)[:, :, :nsample]
    group_first = jnp.broadcast_to(group_idx[:, :, 0:1], group_idx.shape)
    group_idx = jnp.where(group_idx == N, group_first, group_idx)
    return group_idx


def _bn_relu(x, g, b, axes):
    mean = jnp.mean(x, axis=axes, keepdims=True)
    var = jnp.var(x, axis=axes, keepdims=True)
    return jax.nn.relu((x - mean) / jnp.sqrt(var + 1e-5) * g + b)


def _mlp_stack(x, layers, axes):
    for (W, b, g, be) in layers:
        x = x @ W + b
        x = _bn_relu(x, g, be, axes)
    return x


def _sa_msg(feature, coord, n_group, k_list, radius_list, branches):
    xyz = jnp.transpose(coord, (0, 2, 1))
    points = jnp.transpose(feature, (0, 2, 1))
    fps_idx = fps_pallas(coord[:, :3, :], n_group)
    new_xyz = _index_points(xyz, fps_idx)
    outs = []
    for k, radius, layers in zip(k_list, radius_list, branches):
        gidx = _query_ball_point(radius, k, xyz, new_xyz)
        grouped_xyz = _index_points(xyz, gidx) - new_xyz[:, :, None, :]
        grouped = jnp.concatenate([_index_points(points, gidx), grouped_xyz], -1)
        h = _mlp_stack(grouped, layers, axes=(0, 1, 2))
        outs.append(jnp.max(h, axis=2))
    new_points = jnp.concatenate(outs, -1)
    return jnp.transpose(new_points, (0, 2, 1)), jnp.transpose(new_xyz, (0, 2, 1))


def _sa_group_all(feature, coord, layers):
    xyz = jnp.transpose(coord, (0, 2, 1))
    points = jnp.transpose(feature, (0, 2, 1))
    B, N, _ = xyz.shape
    new_xyz = jnp.zeros((B, 1, 3), jnp.float32)
    grouped = jnp.concatenate([points, xyz], -1)[:, None, :, :]
    h = _mlp_stack(grouped, layers, axes=(0, 1, 2))
    new_points = jnp.max(h, axis=2)
    return jnp.transpose(new_points, (0, 2, 1)), jnp.transpose(new_xyz, (0, 2, 1))


def _feature_propagation(points1, xyz1, points2, xyz2, layers):
    x1 = jnp.transpose(xyz1, (0, 2, 1))
    x2 = jnp.transpose(xyz2, (0, 2, 1))
    p2 = jnp.transpose(points2, (0, 2, 1))
    B, N, _ = x1.shape
    S = x2.shape[1]
    if S == 1:
        interp = jnp.broadcast_to(p2, (B, N, p2.shape[-1]))
    else:
        d = _square_distance(x1, x2)
        idx = jnp.argsort(d, axis=-1)[:, :, :3]
        dd = jnp.take_along_axis(d, idx, axis=-1)
        w = 1.0 / (dd + 1e-8)
        w = w / jnp.sum(w, -1, keepdims=True)
        interp = jnp.sum(_index_points(p2, idx) * w[..., None], axis=2)
    new_points = jnp.concatenate([jnp.transpose(points1, (0, 2, 1)), interp], -1)
    h = _mlp_stack(new_points, layers, axes=(0, 1))
    return jnp.transpose(h, (0, 2, 1))


def kernel(coord, label, params):
    n_p = coord.shape[2]
    feature_0 = coord
    coord_0 = coord[:, :3, :]
    f1, c1 = _sa_msg(feature_0, coord_0, 512, [32, 64, 128], [0.1, 0.2, 0.4],
                     params['sa1'])
    f2, c2 = _sa_msg(f1, c1, 128, [64, 128], [0.4, 0.8], params['sa2'])
    f3, c3 = _sa_group_all(f2, c2, params['sa3'])
    f2 = _feature_propagation(f2, c2, f3, c3, params['fp3'])
    f1 = _feature_propagation(f1, c1, f2, c2, params['fp2'])
    lab = jnp.broadcast_to(label[:, :, None], (label.shape[0], label.shape[1], n_p))
    p1 = jnp.concatenate([lab, coord_0, feature_0], 1)
    f0 = _feature_propagation(p1, coord_0, f1, c1, params['fp1'])
    h = jnp.transpose(f0, (0, 2, 1))
    h = _mlp_stack(h, [params['head_mlp']], axes=(0, 1))
    logits = h @ params['head_W'] + params['head_b']
    feature = jax.nn.log_softmax(logits, axis=-1)
    return feature, c3


# speedup vs baseline: 1.0825x; 1.0506x over previous
import functools
import jax
import jax.numpy as jnp
from jax import lax
from jax.experimental import pallas as pl
from jax.experimental.pallas import tpu as pltpu


def _fps_body(xyz_ref, out_ref, *, npoint):
    B, _, N = xyz_ref.shape
    x0 = xyz_ref[:, 0, :]
    x1 = xyz_ref[:, 1, :]
    x2 = xyz_ref[:, 2, :]
    iota_n = (lax.broadcasted_iota(jnp.int32, (B, N), 1)
              + 0 * lax.broadcasted_iota(jnp.int32, (B, N), 0)
              ).astype(jnp.float32)
    iota_s = (lax.broadcasted_iota(jnp.int32, (B, npoint), 1)
              + 0 * lax.broadcasted_iota(jnp.int32, (B, npoint), 0)
              ).astype(jnp.float32)

    def body(i, st):
        cent, dist, far = st
        cent = jnp.where(iota_s == i.astype(jnp.float32), far, cent)
        mask = iota_n == far
        cx = jnp.sum(jnp.where(mask, x0, 0.0), axis=1, keepdims=True)
        cy = jnp.sum(jnp.where(mask, x1, 0.0), axis=1, keepdims=True)
        cz = jnp.sum(jnp.where(mask, x2, 0.0), axis=1, keepdims=True)
        d0 = x0 - cx
        d1 = x1 - cy
        d2 = x2 - cz
        d = d0 * d0 + d1 * d1 + d2 * d2
        dist = jnp.minimum(dist, d)
        m = jnp.max(dist, axis=1, keepdims=True)
        far_new = jnp.min(jnp.where(dist == m, iota_n, float(N)), axis=1,
                          keepdims=True)
        return cent, dist, far_new

    init = (x0[:, :npoint] * 0.0,
            x0 * 0.0 + 1e10,
            x0[:, :1] * 0.0)
    cent, _, _ = lax.fori_loop(0, npoint, body, init)
    out_ref[...] = cent.astype(jnp.int32)


def fps_pallas(xyz_t, npoint):
    B, _, N = xyz_t.shape
    return pl.pallas_call(
        functools.partial(_fps_body, npoint=npoint),
        out_shape=jax.ShapeDtypeStruct((B, npoint), jnp.int32),
    )(xyz_t)


def _ballquery_body(new_ref, xyz_ref, *out_refs, branches, nc):
    _, Sb, _ = new_ref.shape
    N = xyz_ref.shape[1]
    a = new_ref[0]
    x = xyz_ref[0]
    a2 = jnp.sum(a * a, axis=1, keepdims=True)
    x2 = jnp.sum(x * x, axis=1, keepdims=True)
    prod = jax.lax.dot_general(
        a, x, (((1,), (1,)), ((), ())),
        preferred_element_type=jnp.float32)
    d = a2 + x2.T - 2.0 * prod
    for (radius, k), out_ref in zip(branches, out_refs):
        mask = (d <= radius * radius).astype(jnp.float32)
        p = mask
        sh = 1
        while sh < N:
            rolled = pltpu.roll(p, sh, 1)
            lane = lax.broadcasted_iota(jnp.int32, (Sb, N), 1)
            p = p + jnp.where(lane >= sh, rolled, 0.0)
            sh *= 2
        acc = jnp.zeros((Sb, k), jnp.float32)
        jvec = (lax.broadcasted_iota(jnp.int32, (Sb, k, nc), 1)
                ).astype(jnp.float32)
        for t in range(N // nc):
            pc = p[:, t * nc:(t + 1) * nc]
            cmp = (pc[:, None, :] <= jvec).astype(jnp.float32)
            acc = acc + jnp.sum(cmp, axis=2)
        first = acc[:, :1]
        sel = jnp.where(acc >= float(N) - 0.5, first, acc)
        out_ref[0] = sel.astype(jnp.int32)


def ballquery_pallas(new_xyz, xyz, branches, sb=32, nc=128):
    B, S, _ = new_xyz.shape
    N = xyz.shape[1]
    grid = (B, S // sb)
    out_shapes = [jax.ShapeDtypeStruct((B, S, k), jnp.int32)
                  for _, k in branches]
    out_specs = [pl.BlockSpec((1, sb, k), lambda b, s, k=k: (b, s, 0))
                 for _, k in branches]
    return pl.pallas_call(
        functools.partial(_ballquery_body, branches=branches, nc=nc),
        grid=grid,
        in_specs=[pl.BlockSpec((1, sb, 3), lambda b, s: (b, s, 0)),
                  pl.BlockSpec((1, N, 3), lambda b, s: (b, 0, 0))],
        out_specs=out_specs,
        out_shape=out_shapes,
    )(new_xyz, xyz)


def _knn3_body(x1_ref, x2_ref, p2_ref, out_ref):
    _, Nb, _ = x1_ref.shape
    S = x2_ref.shape[1]
    a = x1_ref[0]
    x = x2_ref[0]
    a2 = jnp.sum(a * a, axis=1, keepdims=True)
    x2 = jnp.sum(x * x, axis=1, keepdims=True)
    prod = jax.lax.dot_general(
        a, x, (((1,), (1,)), ((), ())),
        preferred_element_type=jnp.float32)
    d = a2 + x2.T - 2.0 * prod
    iota = (lax.broadcasted_iota(jnp.int32, (Nb, S), 1)
            + 0 * lax.broadcasted_iota(jnp.int32, (Nb, S), 0)
            ).astype(jnp.float32)
    wmat = jnp.zeros((Nb, S), jnp.float32)
    wsum = jnp.zeros((Nb, 1), jnp.float32)
    dcur = d
    onehots = []
    for _ in range(3):
        dj = jnp.min(dcur, axis=1, keepdims=True)
        ij = jnp.min(jnp.where(dcur == dj, iota, float(S)), axis=1,
                     keepdims=True)
        oh = (iota == ij).astype(jnp.float32)
        wj = 1.0 / (dj + 1e-8)
        wsum = wsum + wj
        onehots.append((oh, wj))
        dcur = jnp.where(oh > 0.0, jnp.float32(jnp.inf), dcur)
    for oh, wj in onehots:
        wmat = wmat + oh * (wj / wsum)
    out_ref[0] = jax.lax.dot_general(
        wmat, p2_ref[0], (((1,), (0,)), ((), ())),
        preferred_element_type=jnp.float32)


def knn3_interp_pallas(x1, x2, p2, nb):
    B, N, _ = x1.shape
    S = x2.shape[1]
    C = p2.shape[2]
    grid = (B, N // nb)
    return pl.pallas_call(
        _knn3_body,
        grid=grid,
        in_specs=[pl.BlockSpec((1, nb, 3), lambda b, n: (b, n, 0)),
                  pl.BlockSpec((1, S, 3), lambda b, n: (b, 0, 0)),
                  pl.BlockSpec((1, S, C), lambda b, n: (b, 0, 0))],
        out_specs=pl.BlockSpec((1, nb, C), lambda b, n: (b, n, 0)),
        out_shape=jax.ShapeDtypeStruct((B, N, C), jnp.float32),
    )(x1, x2, p2)


def _square_distance(src, dst):
    return (jnp.sum(src ** 2, -1)[:, :, None] + jnp.sum(dst ** 2, -1)[:, None, :]
            - 2.0 * jnp.einsum('bnc,bmc->bnm', src, dst))


def _index_points(points, idx):
    return jax.vmap(lambda p, i: p[i])(points, idx)


def _query_ball_point(radius, nsample, xyz, new_xyz):
    B, N, _ = xyz.shape
    S = new_xyz.shape[1]
    sqrdists = _square_distance(new_xyz, xyz)
    group_idx = jnp.broadcast_to(jnp.arange(N, dtype=jnp.int32), (B, S, N))
    group_idx = jnp.where(sqrdists > radius ** 2, N, group_idx)
    group_idx = jnp.sort(group_idx, axis=-1)[:, :, :nsample]
    group_first = jnp.broadcast_to(group_idx[:, :, 0:1], group_idx.shape)
    group_idx = jnp.where(group_idx == N, group_first, group_idx)
    return group_idx


def _bn_relu(x, g, b, axes):
    mean = jnp.mean(x, axis=axes, keepdims=True)
    var = jnp.var(x, axis=axes, keepdims=True)
    return jax.nn.relu((x - mean) / jnp.sqrt(var + 1e-5) * g + b)


def _mlp_stack(x, layers, axes):
    for (W, b, g, be) in layers:
        x = x @ W + b
        x = _bn_relu(x, g, be, axes)
    return x


def _sa_msg(feature, coord, n_group, k_list, radius_list, branches):
    xyz = jnp.transpose(coord, (0, 2, 1))
    points = jnp.transpose(feature, (0, 2, 1))
    fps_idx = fps_pallas(coord[:, :3, :], n_group)
    new_xyz = _index_points(xyz, fps_idx)
    gidxs = ballquery_pallas(new_xyz, xyz,
                             [(r, k) for k, r in zip(k_list, radius_list)])
    outs = []
    for k, radius, layers, gidx in zip(k_list, radius_list, branches, gidxs):
        grouped_xyz = _index_points(xyz, gidx) - new_xyz[:, :, None, :]
        grouped = jnp.concatenate([_index_points(points, gidx), grouped_xyz], -1)
        h = _mlp_stack(grouped, layers, axes=(0, 1, 2))
        outs.append(jnp.max(h, axis=2))
    new_points = jnp.concatenate(outs, -1)
    return jnp.transpose(new_points, (0, 2, 1)), jnp.transpose(new_xyz, (0, 2, 1))


def _sa_group_all(feature, coord, layers):
    xyz = jnp.transpose(coord, (0, 2, 1))
    points = jnp.transpose(feature, (0, 2, 1))
    B, N, _ = xyz.shape
    new_xyz = jnp.zeros((B, 1, 3), jnp.float32)
    grouped = jnp.concatenate([points, xyz], -1)[:, None, :, :]
    h = _mlp_stack(grouped, layers, axes=(0, 1, 2))
    new_points = jnp.max(h, axis=2)
    return jnp.transpose(new_points, (0, 2, 1)), jnp.transpose(new_xyz, (0, 2, 1))


def _feature_propagation(points1, xyz1, points2, xyz2, layers):
    x1 = jnp.transpose(xyz1, (0, 2, 1))
    x2 = jnp.transpose(xyz2, (0, 2, 1))
    p2 = jnp.transpose(points2, (0, 2, 1))
    B, N, _ = x1.shape
    S = x2.shape[1]
    if S == 1:
        interp = jnp.broadcast_to(p2, (B, N, p2.shape[-1]))
    else:
        interp = knn3_interp_pallas(x1, x2, p2, nb=min(N, 512))
    new_points = jnp.concatenate([jnp.transpose(points1, (0, 2, 1)), interp], -1)
    h = _mlp_stack(new_points, layers, axes=(0, 1))
    return jnp.transpose(h, (0, 2, 1))


def kernel(coord, label, params):
    n_p = coord.shape[2]
    feature_0 = coord
    coord_0 = coord[:, :3, :]
    f1, c1 = _sa_msg(feature_0, coord_0, 512, [32, 64, 128], [0.1, 0.2, 0.4],
                     params['sa1'])
    f2, c2 = _sa_msg(f1, c1, 128, [64, 128], [0.4, 0.8], params['sa2'])
    f3, c3 = _sa_group_all(f2, c2, params['sa3'])
    f2 = _feature_propagation(f2, c2, f3, c3, params['fp3'])
    f1 = _feature_propagation(f1, c1, f2, c2, params['fp2'])
    lab = jnp.broadcast_to(label[:, :, None], (label.shape[0], label.shape[1], n_p))
    p1 = jnp.concatenate([lab, coord_0, feature_0], 1)
    f0 = _feature_propagation(p1, coord_0, f1, c1, params['fp1'])
    h = jnp.transpose(f0, (0, 2, 1))
    h = _mlp_stack(h, [params['head_mlp']], axes=(0, 1))
    logits = h @ params['head_W'] + params['head_b']
    feature = jax.nn.log_softmax(logits, axis=-1)
    return feature, c3


# speedup vs baseline: 15.9170x; 14.7041x over previous
import functools
import jax
import jax.numpy as jnp
from jax import lax
from jax.experimental import pallas as pl
from jax.experimental.pallas import tpu as pltpu
from jax.experimental.pallas import tpu_sc as plsc


def sc_gather_rows(table, idx):
    T, D = table.shape
    R = idx.shape[0]
    nw = 32
    rows_w = R // nw
    assert rows_w % 128 == 0
    ch = 128
    while ch * 2 <= min(2048, (96 * 1024) // (D * 4), rows_w) and rows_w % (ch * 2) == 0:
        ch *= 2
    nrow = ch // 128
    n_chunks = rows_w // ch
    idx2 = idx.reshape(R // 128, 128)
    mesh = plsc.VectorSubcoreMesh(core_axis_name="c", subcore_axis_name="s")

    @functools.partial(
        pl.kernel, mesh=mesh,
        out_type=jax.ShapeDtypeStruct((R, D), jnp.float32),
        compiler_params=pltpu.CompilerParams(use_tc_tiling_on_sc=False),
        scratch_types=[
            pltpu.VMEM((nrow, 128), jnp.int32),
            pltpu.VMEM((ch, D), jnp.float32),
            pltpu.SemaphoreType.DMA,
        ],
    )
    def gk(idx_hbm, table_hbm, out_hbm, idx_v, rows_v, sem):
        wid = lax.axis_index("s") * 2 + lax.axis_index("c")
        base = wid * rows_w

        def body(t, carry):
            off = base + t * ch
            pltpu.sync_copy(idx_hbm.at[pl.ds(off // 128, nrow)], idx_v)
            handles = []
            for j in range(nrow):
                handles.append(pltpu.async_copy(
                    table_hbm.at[idx_v.at[j]],
                    rows_v.at[pl.ds(j * 128, 128)], sem))
            for h in handles:
                h.wait()
            pltpu.sync_copy(rows_v, out_hbm.at[pl.ds(off, ch)])
            return carry

        lax.fori_loop(0, n_chunks, body, 0)

    return gk(idx2, table)


def _fps_body(xyz_ref, out_ref, *, npoint):
    B, _, N = xyz_ref.shape
    x0 = xyz_ref[:, 0, :]
    x1 = xyz_ref[:, 1, :]
    x2 = xyz_ref[:, 2, :]
    iota_n = (lax.broadcasted_iota(jnp.int32, (B, N), 1)
              + 0 * lax.broadcasted_iota(jnp.int32, (B, N), 0)
              ).astype(jnp.float32)
    iota_s = (lax.broadcasted_iota(jnp.int32, (B, npoint), 1)
              + 0 * lax.broadcasted_iota(jnp.int32, (B, npoint), 0)
              ).astype(jnp.float32)

    def body(i, st):
        cent, dist, far = st
        cent = jnp.where(iota_s == i.astype(jnp.float32), far, cent)
        mask = iota_n == far
        cx = jnp.sum(jnp.where(mask, x0, 0.0), axis=1, keepdims=True)
        cy = jnp.sum(jnp.where(mask, x1, 0.0), axis=1, keepdims=True)
        cz = jnp.sum(jnp.where(mask, x2, 0.0), axis=1, keepdims=True)
        d0 = x0 - cx
        d1 = x1 - cy
        d2 = x2 - cz
        d = d0 * d0 + d1 * d1 + d2 * d2
        dist = jnp.minimum(dist, d)
        m = jnp.max(dist, axis=1, keepdims=True)
        far_new = jnp.min(jnp.where(dist == m, iota_n, float(N)), axis=1,
                          keepdims=True)
        return cent, dist, far_new

    init = (x0[:, :npoint] * 0.0,
            x0 * 0.0 + 1e10,
            x0[:, :1] * 0.0)
    cent, _, _ = lax.fori_loop(0, npoint, body, init)
    out_ref[...] = cent.astype(jnp.int32)


def fps_pallas(xyz_t, npoint):
    B, _, N = xyz_t.shape
    return pl.pallas_call(
        functools.partial(_fps_body, npoint=npoint),
        out_shape=jax.ShapeDtypeStruct((B, npoint), jnp.int32),
    )(xyz_t)


def _ballquery_body(new_ref, xyz_ref, *out_refs, branches, nc):
    _, Sb, _ = new_ref.shape
    N = xyz_ref.shape[1]
    a = new_ref[0]
    x = xyz_ref[0]
    a2 = jnp.sum(a * a, axis=1, keepdims=True)
    x2 = jnp.sum(x * x, axis=1, keepdims=True)
    prod = jax.lax.dot_general(
        a, x, (((1,), (1,)), ((), ())),
        preferred_element_type=jnp.float32)
    d = a2 + x2.T - 2.0 * prod
    for (radius, k), out_ref in zip(branches, out_refs):
        mask = (d <= radius * radius).astype(jnp.float32)
        p = mask
        sh = 1
        while sh < N:
            rolled = pltpu.roll(p, sh, 1)
            lane = lax.broadcasted_iota(jnp.int32, (Sb, N), 1)
            p = p + jnp.where(lane >= sh, rolled, 0.0)
            sh *= 2
        acc = jnp.zeros((Sb, k), jnp.float32)
        jvec = (lax.broadcasted_iota(jnp.int32, (Sb, k, nc), 1)
                ).astype(jnp.float32)
        for t in range(N // nc):
            pc = p[:, t * nc:(t + 1) * nc]
            cmp = (pc[:, None, :] <= jvec).astype(jnp.float32)
            acc = acc + jnp.sum(cmp, axis=2)
        first = acc[:, :1]
        sel = jnp.where(acc >= float(N) - 0.5, first, acc)
        out_ref[0] = sel.astype(jnp.int32)


def ballquery_pallas(new_xyz, xyz, branches, sb=32, nc=128):
    B, S, _ = new_xyz.shape
    N = xyz.shape[1]
    grid = (B, S // sb)
    out_shapes = [jax.ShapeDtypeStruct((B, S, k), jnp.int32)
                  for _, k in branches]
    out_specs = [pl.BlockSpec((1, sb, k), lambda b, s, k=k: (b, s, 0))
                 for _, k in branches]
    return pl.pallas_call(
        functools.partial(_ballquery_body, branches=branches, nc=nc),
        grid=grid,
        in_specs=[pl.BlockSpec((1, sb, 3), lambda b, s: (b, s, 0)),
                  pl.BlockSpec((1, N, 3), lambda b, s: (b, 0, 0))],
        out_specs=out_specs,
        out_shape=out_shapes,
    )(new_xyz, xyz)


def _knn3_body(x1_ref, x2_ref, p2_ref, out_ref):
    _, Nb, _ = x1_ref.shape
    S = x2_ref.shape[1]
    a = x1_ref[0]
    x = x2_ref[0]
    a2 = jnp.sum(a * a, axis=1, keepdims=True)
    x2 = jnp.sum(x * x, axis=1, keepdims=True)
    prod = jax.lax.dot_general(
        a, x, (((1,), (1,)), ((), ())),
        preferred_element_type=jnp.float32)
    d = a2 + x2.T - 2.0 * prod
    iota = (lax.broadcasted_iota(jnp.int32, (Nb, S), 1)
            + 0 * lax.broadcasted_iota(jnp.int32, (Nb, S), 0)
            ).astype(jnp.float32)
    wmat = jnp.zeros((Nb, S), jnp.float32)
    wsum = jnp.zeros((Nb, 1), jnp.float32)
    dcur = d
    onehots = []
    for _ in range(3):
        dj = jnp.min(dcur, axis=1, keepdims=True)
        ij = jnp.min(jnp.where(dcur == dj, iota, float(S)), axis=1,
                     keepdims=True)
        oh = (iota == ij).astype(jnp.float32)
        wj = 1.0 / (dj + 1e-8)
        wsum = wsum + wj
        onehots.append((oh, wj))
        dcur = jnp.where(oh > 0.0, jnp.float32(jnp.inf), dcur)
    for oh, wj in onehots:
        wmat = wmat + oh * (wj / wsum)
    out_ref[0] = jax.lax.dot_general(
        wmat, p2_ref[0], (((1,), (0,)), ((), ())),
        preferred_element_type=jnp.float32)


def knn3_interp_pallas(x1, x2, p2, nb):
    B, N, _ = x1.shape
    S = x2.shape[1]
    C = p2.shape[2]
    grid = (B, N // nb)
    return pl.pallas_call(
        _knn3_body,
        grid=grid,
        in_specs=[pl.BlockSpec((1, nb, 3), lambda b, n: (b, n, 0)),
                  pl.BlockSpec((1, S, 3), lambda b, n: (b, 0, 0)),
                  pl.BlockSpec((1, S, C), lambda b, n: (b, 0, 0))],
        out_specs=pl.BlockSpec((1, nb, C), lambda b, n: (b, n, 0)),
        out_shape=jax.ShapeDtypeStruct((B, N, C), jnp.float32),
    )(x1, x2, p2)


def _square_distance(src, dst):
    return (jnp.sum(src ** 2, -1)[:, :, None] + jnp.sum(dst ** 2, -1)[:, None, :]
            - 2.0 * jnp.einsum('bnc,bmc->bnm', src, dst))


def _index_points(points, idx):
    return jax.vmap(lambda p, i: p[i])(points, idx)


def _query_ball_point(radius, nsample, xyz, new_xyz):
    B, N, _ = xyz.shape
    S = new_xyz.shape[1]
    sqrdists = _square_distance(new_xyz, xyz)
    group_idx = jnp.broadcast_to(jnp.arange(N, dtype=jnp.int32), (B, S, N))
    group_idx = jnp.where(sqrdists > radius ** 2, N, group_idx)
    group_idx = jnp.sort(group_idx, axis=-1)[:, :, :nsample]
    group_first = jnp.broadcast_to(group_idx[:, :, 0:1], group_idx.shape)
    group_idx = jnp.where(group_idx == N, group_first, group_idx)
    return group_idx


def _bn_relu(x, g, b, axes):
    mean = jnp.mean(x, axis=axes, keepdims=True)
    var = jnp.var(x, axis=axes, keepdims=True)
    return jax.nn.relu((x - mean) / jnp.sqrt(var + 1e-5) * g + b)


def _mlp_stack(x, layers, axes):
    for (W, b, g, be) in layers:
        x = x @ W + b
        x = _bn_relu(x, g, be, axes)
    return x


def _sa_msg(feature, coord, n_group, k_list, radius_list, branches):
    xyz = jnp.transpose(coord, (0, 2, 1))
    points = jnp.transpose(feature, (0, 2, 1))
    B, N, _ = xyz.shape
    S = n_group
    Cp = points.shape[2]
    dpad = -(-(Cp + 3) // 16) * 16
    table = jnp.concatenate(
        [points, xyz,
         jnp.zeros((B, N, dpad - Cp - 3), jnp.float32)], -1
    ).reshape(B * N, dpad)
    fps_idx = fps_pallas(coord[:, :3, :], n_group)
    new_xyz = _index_points(xyz, fps_idx)
    gidxs = ballquery_pallas(new_xyz, xyz,
                             [(r, k) for k, r in zip(k_list, radius_list)])
    offs = (jnp.arange(B, dtype=jnp.int32) * N)[:, None, None]
    outs = []
    for k, radius, layers, gidx in zip(k_list, radius_list, branches, gidxs):
        flat = (jnp.minimum(gidx, N - 1) + offs).reshape(-1)
        rows = sc_gather_rows(table, flat).reshape(B, S, k, dpad)
        grouped_xyz = rows[..., Cp:Cp + 3] - new_xyz[:, :, None, :]
        grouped = jnp.concatenate([rows[..., :Cp], grouped_xyz], -1)
        h = _mlp_stack(grouped, layers, axes=(0, 1, 2))
        outs.append(jnp.max(h, axis=2))
    new_points = jnp.concatenate(outs, -1)
    return jnp.transpose(new_points, (0, 2, 1)), jnp.transpose(new_xyz, (0, 2, 1))


def _sa_group_all(feature, coord, layers):
    xyz = jnp.transpose(coord, (0, 2, 1))
    points = jnp.transpose(feature, (0, 2, 1))
    B, N, _ = xyz.shape
    new_xyz = jnp.zeros((B, 1, 3), jnp.float32)
    grouped = jnp.concatenate([points, xyz], -1)[:, None, :, :]
    h = _mlp_stack(grouped, layers, axes=(0, 1, 2))
    new_points = jnp.max(h, axis=2)
    return jnp.transpose(new_points, (0, 2, 1)), jnp.transpose(new_xyz, (0, 2, 1))


def _feature_propagation(points1, xyz1, points2, xyz2, layers):
    x1 = jnp.transpose(xyz1, (0, 2, 1))
    x2 = jnp.transpose(xyz2, (0, 2, 1))
    p2 = jnp.transpose(points2, (0, 2, 1))
    B, N, _ = x1.shape
    S = x2.shape[1]
    if S == 1:
        interp = jnp.broadcast_to(p2, (B, N, p2.shape[-1]))
    else:
        interp = knn3_interp_pallas(x1, x2, p2, nb=min(N, 512))
    new_points = jnp.concatenate([jnp.transpose(points1, (0, 2, 1)), interp], -1)
    h = _mlp_stack(new_points, layers, axes=(0, 1))
    return jnp.transpose(h, (0, 2, 1))


def kernel(coord, label, params):
    n_p = coord.shape[2]
    feature_0 = coord
    coord_0 = coord[:, :3, :]
    f1, c1 = _sa_msg(feature_0, coord_0, 512, [32, 64, 128], [0.1, 0.2, 0.4],
                     params['sa1'])
    f2, c2 = _sa_msg(f1, c1, 128, [64, 128], [0.4, 0.8], params['sa2'])
    f3, c3 = _sa_group_all(f2, c2, params['sa3'])
    f2 = _feature_propagation(f2, c2, f3, c3, params['fp3'])
    f1 = _feature_propagation(f1, c1, f2, c2, params['fp2'])
    lab = jnp.broadcast_to(label[:, :, None], (label.shape[0], label.shape[1], n_p))
    p1 = jnp.concatenate([lab, coord_0, feature_0], 1)
    f0 = _feature_propagation(p1, coord_0, f1, c1, params['fp1'])
    h = jnp.transpose(f0, (0, 2, 1))
    h = _mlp_stack(h, [params['head_mlp']], axes=(0, 1))
    logits = h @ params['head_W'] + params['head_b']
    feature = jax.nn.log_softmax(logits, axis=-1)
    return feature, c3
